# Initial kernel scaffold; baseline (speedup 1.0000x reference)
import jax
import jax.numpy as jnp
from jax.experimental import pallas as pl


def kernel(x, edge_index, lin1_W, lin1_b, gat_W, att_src, att_dst, gat_b, Wih, Whh, lin2_W, lin2_b):
    raise NotImplementedError("write your pallas kernel here")


# speedup vs baseline: 18.9791x; 18.9791x over previous
import functools

import jax
import jax.numpy as jnp
from jax import lax
from jax.experimental import pallas as pl
from jax.experimental.pallas import tpu as pltpu
from jax.experimental.pallas import tpu_sc as plsc

N, E, D, L = 10000, 320000, 128, 3
RESIDUAL_WEIGHT = 0.1
NE = E + N
C = 96
NTILES = 32
EP = ((NE + NTILES * C - 1) // (NTILES * C)) * (NTILES * C)
EPT = EP // NTILES
NCH = EPT // C
WIDE = 144
NP = 10240
RPT = NP // 16
BN = 1000
GRID = N // BN
_RCHUNKS = [(t * C, C) for t in range(RPT // C)] + [(RPT // C * C, RPT % C)]


def _sc_edge_body(xt_hbm, as_hbm, ad_hbm, src_hbm, dst_hbm, out_hbm,
                  as_t, ad_t, sidx, didx, rows, wbuf, table, sem):
    cid = lax.axis_index("c")
    sid = lax.axis_index("s")
    wid = cid * 16 + sid

    pltpu.sync_copy(as_hbm, as_t)
    pltpu.sync_copy(ad_hbm, ad_t)

    z16 = jnp.zeros((16,), jnp.float32)

    def _zrow(i, _):
        for k in range(WIDE // 16):
            rows[i, pl.ds(k * 16, 16)] = z16
        return 0

    lax.fori_loop(0, C, _zrow, 0)
    r0 = sid * RPT
    for off, size in _RCHUNKS:
        pltpu.sync_copy(rows.at[pl.ds(0, size)],
                        table.at[pl.ds(r0 + off, size)])
    plsc.subcore_barrier()

    base = wid * EPT
    lanes = lax.iota(jnp.int32, 16)

    def _chunk(ci, _):
        off = base + ci * C
        pltpu.sync_copy(src_hbm.at[pl.ds(off, C)], sidx)
        pltpu.sync_copy(dst_hbm.at[pl.ds(off, C)], didx)
        pltpu.async_copy(xt_hbm.at[sidx], rows, sem).wait()
        for j in range(C // 16):
            s_ids = sidx[pl.ds(j * 16, 16)]
            d_ids = didx[pl.ds(j * 16, 16)]
            e = plsc.load_gather(as_t, [s_ids]) + plsc.load_gather(ad_t, [d_ids])
            e = jnp.where(e >= 0.0, e, 0.2 * e)
            w = jnp.exp(e)
            gi = off + j * 16 + lanes
            w = jnp.where(gi < NE, w, 0.0)
            wbuf[pl.ds(j * 16, 16)] = w

        def _srow(r, _):
            wv = plsc.load_gather(wbuf, [jnp.full((16,), r, jnp.int32)])
            for k in range(WIDE // 16):
                rows[r, pl.ds(k * 16, 16)] = rows[r, pl.ds(k * 16, 16)] * wv
            return 0

        lax.fori_loop(0, C, _srow, 0)
        pltpu.sync_copy(rows, table.at[didx], add=True)
        return 0

    lax.fori_loop(0, NCH, _chunk, 0)
    plsc.subcore_barrier()

    for off, size in _RCHUNKS:
        pltpu.sync_copy(table.at[pl.ds(r0 + off, size)],
                        out_hbm.at[cid, pl.ds(r0 + off, size)])


@functools.cache
def _sc_edge():
    return pl.kernel(
        _sc_edge_body,
        out_type=jax.ShapeDtypeStruct((2, NP, WIDE), jnp.float32),
        mesh=plsc.VectorSubcoreMesh(core_axis_name="c", subcore_axis_name="s"),
        compiler_params=pltpu.CompilerParams(needs_layout_passes=False,
                                             use_tc_tiling_on_sc=False),
        scratch_types=[
            pltpu.VMEM((N,), jnp.float32),
            pltpu.VMEM((N,), jnp.float32),
            pltpu.VMEM((C,), jnp.int32),
            pltpu.VMEM((C,), jnp.int32),
            pltpu.VMEM((C, WIDE), jnp.float32),
            pltpu.VMEM((C,), jnp.float32),
            pltpu.VMEM_SHARED((NP, WIDE), jnp.float32),
            pltpu.SemaphoreType.DMA,
        ],
    )


def _full(shape):
    return pl.BlockSpec(shape, lambda i: (0,) * len(shape))


def _rows(width):
    return pl.BlockSpec((BN, width), lambda i: (i, 0))


def _prelude_body(x_ref, w1t, b1, w0t, e1, a0, x1_o, xt_o, aux_o):
    xr = jnp.dot(x_ref[...], w1t[...], preferred_element_type=jnp.float32) + b1[...]
    x1_o[...] = xr
    xt = jnp.dot(xr, w0t[...], preferred_element_type=jnp.float32) + e1[...]
    xt_o[...] = xt
    aux_o[...] = jnp.dot(xt, a0[...], preferred_element_type=jnp.float32)


_tc_prelude = pl.pallas_call(
    _prelude_body,
    grid=(GRID,),
    in_specs=[_rows(D), _full((D, D)), _full((1, D)), _full((D, WIDE)),
              _full((1, WIDE)), _full((WIDE, D))],
    out_specs=[_rows(D), _rows(WIDE), _rows(D)],
    out_shape=[jax.ShapeDtypeStruct((N, D), jnp.float32),
               jax.ShapeDtypeStruct((N, WIDE), jnp.float32),
               jax.ShapeDtypeStruct((N, D), jnp.float32)],
)


def _lstm(wa, wb, gb, h, c, inp, wih_t, whh_t):
    den = wa[:, D:D + 1] + wb[:, D:D + 1]
    num = wa[:, :D] + wb[:, :D]
    xb = jnp.tanh(num / den + gb)
    g = (jnp.dot(xb, wih_t, preferred_element_type=jnp.float32)
         + jnp.dot(h, whh_t, preferred_element_type=jnp.float32))
    i = jax.nn.sigmoid(g[:, 0:D])
    f = jax.nn.sigmoid(g[:, D:2 * D])
    gg = jnp.tanh(g[:, 2 * D:3 * D])
    o = jax.nn.sigmoid(g[:, 3 * D:4 * D])
    cn = f * c + i * gg
    hn = o * jnp.tanh(cn)
    return hn, cn, hn + RESIDUAL_WEIGHT * inp


def _layer_body(wa, wb, gb, h, c, inp, wih_t, whh_t, wnt, e1, an,
                h_o, c_o, x_o, xt_o, aux_o):
    hn, cn, xn = _lstm(wa[...], wb[...], gb[...], h[...], c[...], inp[...],
                       wih_t[...], whh_t[...])
    h_o[...] = hn
    c_o[...] = cn
    x_o[...] = xn
    xt = jnp.dot(xn, wnt[...], preferred_element_type=jnp.float32) + e1[...]
    xt_o[...] = xt
    aux_o[...] = jnp.dot(xt, an[...], preferred_element_type=jnp.float32)


_tc_layer = pl.pallas_call(
    _layer_body,
    grid=(GRID,),
    in_specs=[_rows(WIDE), _rows(WIDE), _full((1, D)), _rows(D), _rows(D),
              _rows(D), _full((D, 4 * D)), _full((D, 4 * D)),
              _full((D, WIDE)), _full((1, WIDE)), _full((WIDE, D))],
    out_specs=[_rows(D), _rows(D), _rows(D), _rows(WIDE), _rows(D)],
    out_shape=[jax.ShapeDtypeStruct((N, D), jnp.float32),
               jax.ShapeDtypeStruct((N, D), jnp.float32),
               jax.ShapeDtypeStruct((N, D), jnp.float32),
               jax.ShapeDtypeStruct((N, WIDE), jnp.float32),
               jax.ShapeDtypeStruct((N, D), jnp.float32)],
)


def _final_body(wa, wb, gb, h, c, inp, wih_t, whh_t, w2t, b2, out_o):
    _, _, xn = _lstm(wa[...], wb[...], gb[...], h[...], c[...], inp[...],
                     wih_t[...], whh_t[...])
    out_o[...] = (jnp.dot(xn, w2t[...], preferred_element_type=jnp.float32)
                  + b2[...])


_tc_final = pl.pallas_call(
    _final_body,
    grid=(GRID,),
    in_specs=[_rows(WIDE), _rows(WIDE), _full((1, D)), _rows(D), _rows(D),
              _rows(D), _full((D, 4 * D)), _full((D, 4 * D)),
              _full((D, D)), _full((1, D))],
    out_specs=_rows(D),
    out_shape=jax.ShapeDtypeStruct((N, D), jnp.float32),
)


def kernel(x, edge_index, lin1_W, lin1_b, gat_W, att_src, att_dst, gat_b,
           Wih, Whh, lin2_W, lin2_b):
    loops = jnp.arange(N, dtype=edge_index.dtype)
    pad = jnp.zeros((EP - NE,), edge_index.dtype)
    src = jnp.concatenate([edge_index[0], loops, pad])
    dst = jnp.concatenate([edge_index[1], loops, pad])

    W144 = jnp.zeros((L, D, WIDE), jnp.float32)
    W144 = W144.at[:, :, :D].set(jnp.transpose(gat_W, (0, 2, 1)))
    e1 = jnp.zeros((1, WIDE), jnp.float32).at[0, D].set(1.0)
    A = jnp.zeros((L, WIDE, D), jnp.float32)
    A = A.at[:, :D, 0].set(att_src).at[:, :D, 1].set(att_dst)
    Wih_t = jnp.transpose(Wih, (0, 2, 1))
    Whh_t = jnp.transpose(Whh, (0, 2, 1))

    inp, xt, aux = _tc_prelude(x, lin1_W.T, lin1_b.reshape(1, D),
                               W144[0], e1, A[0])
    h = jnp.zeros((N, D), jnp.float32)
    c = jnp.zeros((N, D), jnp.float32)
    for l in range(L):
        wides = _sc_edge()(xt, aux[:, 0], aux[:, 1], src, dst)[:, :N]
        gb = gat_b[l].reshape(1, D)
        if l < L - 1:
            h, c, _x, xt, aux = _tc_layer(
                wides[0], wides[1], gb, h, c, inp, Wih_t[l], Whh_t[l],
                W144[l + 1], e1, A[l + 1])
        else:
            return _tc_final(wides[0], wides[1], gb, h, c, inp,
                             Wih_t[l], Whh_t[l], lin2_W.T,
                             lin2_b.reshape(1, D))


# speedup vs baseline: 23.5657x; 1.2417x over previous
import functools

import jax
import jax.numpy as jnp
from jax import lax
from jax.experimental import pallas as pl
from jax.experimental.pallas import tpu as pltpu
from jax.experimental.pallas import tpu_sc as plsc

N, E, D, L = 10000, 320000, 128, 3
RESIDUAL_WEIGHT = 0.1
NE = E + N
C = 64
NTILES = 32
EP = ((NE + NTILES * C - 1) // (NTILES * C)) * (NTILES * C)
EPT = EP // NTILES
NCH = EPT // C
WIDE = 144
NP = 10240
RPT = NP // 16
BN = 1000
GRID = N // BN
_RCHUNKS = [(t * C, C) for t in range(RPT // C)]
if RPT % C:
    _RCHUNKS.append((RPT // C * C, RPT % C))


def _sc_edge_body(xt_hbm, as_hbm, ad_hbm, src_hbm, dst_hbm, out_hbm,
                  as_t, ad_t, sidx, didx, rows, wbuf, table, sem0, sem1):
    cid = lax.axis_index("c")
    sid = lax.axis_index("s")
    wid = cid * 16 + sid

    pltpu.sync_copy(as_hbm, as_t)
    pltpu.sync_copy(ad_hbm, ad_t)

    z16 = jnp.zeros((16,), jnp.float32)

    def _zrow(i, _):
        for k in range(WIDE // 16):
            rows[0, i, pl.ds(k * 16, 16)] = z16
        return 0

    lax.fori_loop(0, C, _zrow, 0)
    r0 = sid * RPT
    for off, size in _RCHUNKS:
        pltpu.sync_copy(rows.at[0, pl.ds(0, size)],
                        table.at[pl.ds(r0 + off, size)])
    plsc.subcore_barrier()

    base = wid * EPT
    lanes = lax.iota(jnp.int32, 16)
    sems = (sem0, sem1)

    def _fetch(g, b):
        off = base + g * C
        pltpu.sync_copy(src_hbm.at[pl.ds(off, C)], sidx.at[b])
        pltpu.sync_copy(dst_hbm.at[pl.ds(off, C)], didx.at[b])
        pltpu.async_copy(xt_hbm.at[sidx.at[b]], rows.at[b], sems[b])

    _fetch(0, 0)
    _fetch(1, 1)

    def _outer(gg, _):
        for b in range(2):
            g = gg * 2 + b
            off = base + g * C
            pltpu.make_async_copy(xt_hbm.at[sidx.at[b]], rows.at[b],
                                  sems[b]).wait()
            for j in range(C // 16):
                s_ids = sidx[b, pl.ds(j * 16, 16)]
                d_ids = didx[b, pl.ds(j * 16, 16)]
                e = (plsc.load_gather(as_t, [s_ids])
                     + plsc.load_gather(ad_t, [d_ids]))
                e = jnp.where(e >= 0.0, e, 0.2 * e)
                w = jnp.exp(e)
                gi = off + j * 16 + lanes
                w = jnp.where(gi < NE, w, 0.0)
                wbuf[pl.ds(j * 16, 16)] = w

            def _srow(r, _):
                wv = plsc.load_gather(wbuf, [jnp.full((16,), r, jnp.int32)])
                for k in range(D // 16):
                    rows[b, r, pl.ds(k * 16, 16)] = (
                        rows[b, r, pl.ds(k * 16, 16)] * wv)
                rows[b, r, pl.ds(D, 16)] = wv
                return 0

            lax.fori_loop(0, C, _srow, 0)
            pltpu.sync_copy(rows.at[b], table.at[didx.at[b]], add=True)

            nxt = g + 2

            @pl.when(nxt < NCH)
            def _():
                _fetch(nxt, b)

        return 0

    lax.fori_loop(0, NCH // 2, _outer, 0)
    plsc.subcore_barrier()

    for off, size in _RCHUNKS:
        pltpu.sync_copy(table.at[pl.ds(r0 + off, size)],
                        out_hbm.at[cid, pl.ds(r0 + off, size)])


@functools.cache
def _sc_edge():
    return pl.kernel(
        _sc_edge_body,
        out_type=jax.ShapeDtypeStruct((2, NP, WIDE), jnp.float32),
        mesh=plsc.VectorSubcoreMesh(core_axis_name="c", subcore_axis_name="s"),
        compiler_params=pltpu.CompilerParams(needs_layout_passes=False,
                                             use_tc_tiling_on_sc=False),
        scratch_types=[
            pltpu.VMEM((N,), jnp.float32),
            pltpu.VMEM((N,), jnp.float32),
            pltpu.VMEM((2, C), jnp.int32),
            pltpu.VMEM((2, C), jnp.int32),
            pltpu.VMEM((2, C, WIDE), jnp.float32),
            pltpu.VMEM((C,), jnp.float32),
            pltpu.VMEM_SHARED((NP, WIDE), jnp.float32),
            pltpu.SemaphoreType.DMA,
            pltpu.SemaphoreType.DMA,
        ],
    )


def _full(shape):
    return pl.BlockSpec(shape, lambda i: (0,) * len(shape))


def _rows(width):
    return pl.BlockSpec((BN, width), lambda i: (i, 0))


def _prelude_body(x_ref, w1t, b1, w0t, e1, a0, x1_o, xt_o, aux_o):
    xr = jnp.dot(x_ref[...], w1t[...], preferred_element_type=jnp.float32) + b1[...]
    x1_o[...] = xr
    xt = jnp.dot(xr, w0t[...], preferred_element_type=jnp.float32) + e1[...]
    xt_o[...] = xt
    aux_o[...] = jnp.dot(xt, a0[...], preferred_element_type=jnp.float32)


_tc_prelude = pl.pallas_call(
    _prelude_body,
    grid=(GRID,),
    in_specs=[_rows(D), _full((D, D)), _full((1, D)), _full((D, WIDE)),
              _full((1, WIDE)), _full((WIDE, D))],
    out_specs=[_rows(D), _rows(WIDE), _rows(D)],
    out_shape=[jax.ShapeDtypeStruct((N, D), jnp.float32),
               jax.ShapeDtypeStruct((N, WIDE), jnp.float32),
               jax.ShapeDtypeStruct((N, D), jnp.float32)],
)


def _lstm(wa, wb, gb, h, c, inp, wih_t, whh_t):
    den = wa[:, D:D + 1] + wb[:, D:D + 1]
    num = wa[:, :D] + wb[:, :D]
    xb = jnp.tanh(num / den + gb)
    g = (jnp.dot(xb, wih_t, preferred_element_type=jnp.float32)
         + jnp.dot(h, whh_t, preferred_element_type=jnp.float32))
    i = jax.nn.sigmoid(g[:, 0:D])
    f = jax.nn.sigmoid(g[:, D:2 * D])
    gg = jnp.tanh(g[:, 2 * D:3 * D])
    o = jax.nn.sigmoid(g[:, 3 * D:4 * D])
    cn = f * c + i * gg
    hn = o * jnp.tanh(cn)
    return hn, cn, hn + RESIDUAL_WEIGHT * inp


def _layer_body(wa, wb, gb, h, c, inp, wih_t, whh_t, wnt, e1, an,
                h_o, c_o, x_o, xt_o, aux_o):
    hn, cn, xn = _lstm(wa[...], wb[...], gb[...], h[...], c[...], inp[...],
                       wih_t[...], whh_t[...])
    h_o[...] = hn
    c_o[...] = cn
    x_o[...] = xn
    xt = jnp.dot(xn, wnt[...], preferred_element_type=jnp.float32) + e1[...]
    xt_o[...] = xt
    aux_o[...] = jnp.dot(xt, an[...], preferred_element_type=jnp.float32)


_tc_layer = pl.pallas_call(
    _layer_body,
    grid=(GRID,),
    in_specs=[_rows(WIDE), _rows(WIDE), _full((1, D)), _rows(D), _rows(D),
              _rows(D), _full((D, 4 * D)), _full((D, 4 * D)),
              _full((D, WIDE)), _full((1, WIDE)), _full((WIDE, D))],
    out_specs=[_rows(D), _rows(D), _rows(D), _rows(WIDE), _rows(D)],
    out_shape=[jax.ShapeDtypeStruct((N, D), jnp.float32),
               jax.ShapeDtypeStruct((N, D), jnp.float32),
               jax.ShapeDtypeStruct((N, D), jnp.float32),
               jax.ShapeDtypeStruct((N, WIDE), jnp.float32),
               jax.ShapeDtypeStruct((N, D), jnp.float32)],
)


def _final_body(wa, wb, gb, h, c, inp, wih_t, whh_t, w2t, b2, out_o):
    _, _, xn = _lstm(wa[...], wb[...], gb[...], h[...], c[...], inp[...],
                     wih_t[...], whh_t[...])
    out_o[...] = (jnp.dot(xn, w2t[...], preferred_element_type=jnp.float32)
                  + b2[...])


_tc_final = pl.pallas_call(
    _final_body,
    grid=(GRID,),
    in_specs=[_rows(WIDE), _rows(WIDE), _full((1, D)), _rows(D), _rows(D),
              _rows(D), _full((D, 4 * D)), _full((D, 4 * D)),
              _full((D, D)), _full((1, D))],
    out_specs=_rows(D),
    out_shape=jax.ShapeDtypeStruct((N, D), jnp.float32),
)


def kernel(x, edge_index, lin1_W, lin1_b, gat_W, att_src, att_dst, gat_b,
           Wih, Whh, lin2_W, lin2_b):
    loops = jnp.arange(N, dtype=edge_index.dtype)
    pad = jnp.zeros((EP - NE,), edge_index.dtype)
    src = jnp.concatenate([edge_index[0], loops, pad])
    dst = jnp.concatenate([edge_index[1], loops, pad])

    W144 = jnp.zeros((L, D, WIDE), jnp.float32)
    W144 = W144.at[:, :, :D].set(jnp.transpose(gat_W, (0, 2, 1)))
    e1 = jnp.zeros((1, WIDE), jnp.float32).at[0, D].set(1.0)
    A = jnp.zeros((L, WIDE, D), jnp.float32)
    A = A.at[:, :D, 0].set(att_src).at[:, :D, 1].set(att_dst)
    Wih_t = jnp.transpose(Wih, (0, 2, 1))
    Whh_t = jnp.transpose(Whh, (0, 2, 1))

    inp, xt, aux = _tc_prelude(x, lin1_W.T, lin1_b.reshape(1, D),
                               W144[0], e1, A[0])
    h = jnp.zeros((N, D), jnp.float32)
    c = jnp.zeros((N, D), jnp.float32)
    for l in range(L):
        wides = _sc_edge()(xt, aux[:, 0], aux[:, 1], src, dst)[:, :N]
        gb = gat_b[l].reshape(1, D)
        if l < L - 1:
            h, c, _x, xt, aux = _tc_layer(
                wides[0], wides[1], gb, h, c, inp, Wih_t[l], Whh_t[l],
                W144[l + 1], e1, A[l + 1])
        else:
            return _tc_final(wides[0], wides[1], gb, h, c, inp,
                             Wih_t[l], Whh_t[l], lin2_W.T,
                             lin2_b.reshape(1, D))


# speedup vs baseline: 27.1238x; 1.1510x over previous
import functools

import jax
import jax.numpy as jnp
from jax import lax
from jax.experimental import pallas as pl
from jax.experimental.pallas import tpu as pltpu
from jax.experimental.pallas import tpu_sc as plsc

N, E, D, L = 10000, 320000, 128, 3
RESIDUAL_WEIGHT = 0.1
NE = E + N
S = 32
NSUB = 3
C = S * NSUB
NTILES = 32
EP = ((NE + NTILES * C - 1) // (NTILES * C)) * (NTILES * C)
EPT = EP // NTILES
NCH = EPT // C
WIDE = 144
NP = 10240
RPT = NP // 16
BN = 1000
GRID = N // BN
_RCHUNKS = [(t * S, S) for t in range(RPT // S)]
if RPT % S:
    _RCHUNKS.append((RPT // S * S, RPT % S))


def _sc_edge_body(xt_hbm, as_hbm, ad_hbm, idx_hbm, out_hbm,
                  as_t, ad_t, idx, rows, wbuf, table,
                  gsem0, gsem1, gsem2, ssem0, ssem1, ssem2):
    cid = lax.axis_index("c")
    sid = lax.axis_index("s")
    wid = cid * 16 + sid
    gsem = (gsem0, gsem1, gsem2)
    ssem = (ssem0, ssem1, ssem2)

    pltpu.sync_copy(as_hbm, as_t)
    pltpu.sync_copy(ad_hbm, ad_t)

    z16 = jnp.zeros((16,), jnp.float32)

    def _zrow(i, _):
        for k in range(WIDE // 16):
            rows[0, i, pl.ds(k * 16, 16)] = z16
        return 0

    lax.fori_loop(0, S, _zrow, 0)
    r0 = sid * RPT
    for off, size in _RCHUNKS:
        pltpu.sync_copy(rows.at[0, pl.ds(0, size)],
                        table.at[pl.ds(r0 + off, size)])
    plsc.subcore_barrier()

    kk0 = wid * NCH
    base = wid * EPT
    lanes = lax.iota(jnp.int32, 16)

    def _gissue(p, s):
        pltpu.async_copy(xt_hbm.at[idx.at[p, 0, s]], rows.at[s], gsem[s])

    def _gwait(p, s):
        pltpu.make_async_copy(xt_hbm.at[idx.at[p, 0, s]], rows.at[s],
                              gsem[s]).wait()

    def _sissue(p, s):
        pltpu.async_copy(rows.at[s], table.at[idx.at[p, 1, s]], ssem[s],
                         add=True)

    def _swait(p, s):
        pltpu.make_async_copy(rows.at[s], table.at[idx.at[p, 1, s]],
                              ssem[s]).wait()

    def _proc(ci, p, s):
        ebase = base + ci * C + s * S
        for j in range(S // 16):
            s_ids = idx[p, 0, s, pl.ds(j * 16, 16)]
            d_ids = idx[p, 1, s, pl.ds(j * 16, 16)]
            e = (plsc.load_gather(as_t, [s_ids])
                 + plsc.load_gather(ad_t, [d_ids]))
            e = jnp.where(e >= 0.0, e, 0.2 * e)
            w = jnp.exp(e)
            w = jnp.where(ebase + j * 16 + lanes < NE, w, 0.0)
            wbuf[pl.ds(j * 16, 16)] = w

        def _srow(r, _):
            wv = plsc.load_gather(wbuf, [jnp.full((16,), r, jnp.int32)])
            for k in range(D // 16):
                rows[s, r, pl.ds(k * 16, 16)] = (
                    rows[s, r, pl.ds(k * 16, 16)] * wv)
            rows[s, r, pl.ds(D, 16)] = wv
            return 0

        lax.fori_loop(0, S, _srow, 0)

    def _chunk(ci, p, first=False, last=False):
        _gwait(p, 0)
        _proc(ci, p, 0)
        _sissue(p, 0)
        if not first:
            _swait(1 - p, 2)
        if not last:
            pltpu.sync_copy(idx_hbm.at[kk0 + ci + 1], idx.at[1 - p])
        _gissue(p, 2)
        _gwait(p, 1)
        _proc(ci, p, 1)
        _sissue(p, 1)
        _swait(p, 0)
        if not last:
            _gissue(1 - p, 0)
        _gwait(p, 2)
        _proc(ci, p, 2)
        _sissue(p, 2)
        _swait(p, 1)
        if not last:
            _gissue(1 - p, 1)

    pltpu.sync_copy(idx_hbm.at[kk0], idx.at[0])
    _gissue(0, 0)
    _gissue(0, 1)
    _chunk(0, 0, first=True)

    def _outer(cc, _):
        ci = 1 + 2 * cc
        _chunk(ci, 1)
        _chunk(ci + 1, 0)
        return 0

    lax.fori_loop(0, (NCH - 2) // 2, _outer, 0)
    _chunk(NCH - 1, 1, last=True)
    _swait(1, 2)
    plsc.subcore_barrier()

    for off, size in _RCHUNKS:
        pltpu.sync_copy(table.at[pl.ds(r0 + off, size)],
                        out_hbm.at[cid, pl.ds(r0 + off, size)])


@functools.cache
def _sc_edge():
    return pl.kernel(
        _sc_edge_body,
        out_type=jax.ShapeDtypeStruct((2, NP, WIDE), jnp.float32),
        mesh=plsc.VectorSubcoreMesh(core_axis_name="c", subcore_axis_name="s"),
        compiler_params=pltpu.CompilerParams(needs_layout_passes=False,
                                             use_tc_tiling_on_sc=False),
        scratch_types=[
            pltpu.VMEM((N,), jnp.float32),
            pltpu.VMEM((N,), jnp.float32),
            pltpu.VMEM((2, 2, NSUB, S), jnp.int32),
            pltpu.VMEM((NSUB, S, WIDE), jnp.float32),
            pltpu.VMEM((S,), jnp.float32),
            pltpu.VMEM_SHARED((NP, WIDE), jnp.float32),
            pltpu.SemaphoreType.DMA,
            pltpu.SemaphoreType.DMA,
            pltpu.SemaphoreType.DMA,
            pltpu.SemaphoreType.DMA,
            pltpu.SemaphoreType.DMA,
            pltpu.SemaphoreType.DMA,
        ],
    )


def _full(shape):
    return pl.BlockSpec(shape, lambda i: (0,) * len(shape))


def _rows(width):
    return pl.BlockSpec((BN, width), lambda i: (i, 0))


def _prelude_body(x_ref, w1t, b1, w0t, e1, a0, x1_o, xt_o, aux_o):
    xr = jnp.dot(x_ref[...], w1t[...], preferred_element_type=jnp.float32) + b1[...]
    x1_o[...] = xr
    xt = jnp.dot(xr, w0t[...], preferred_element_type=jnp.float32) + e1[...]
    xt_o[...] = xt
    aux_o[...] = jnp.dot(xt, a0[...], preferred_element_type=jnp.float32)


_tc_prelude = pl.pallas_call(
    _prelude_body,
    grid=(GRID,),
    in_specs=[_rows(D), _full((D, D)), _full((1, D)), _full((D, WIDE)),
              _full((1, WIDE)), _full((WIDE, D))],
    out_specs=[_rows(D), _rows(WIDE), _rows(D)],
    out_shape=[jax.ShapeDtypeStruct((N, D), jnp.float32),
               jax.ShapeDtypeStruct((N, WIDE), jnp.float32),
               jax.ShapeDtypeStruct((N, D), jnp.float32)],
)


def _lstm(wa, wb, gb, h, c, inp, wih_t, whh_t):
    den = wa[:, D:D + 1] + wb[:, D:D + 1]
    num = wa[:, :D] + wb[:, :D]
    xb = jnp.tanh(num / den + gb)
    g = (jnp.dot(xb, wih_t, preferred_element_type=jnp.float32)
         + jnp.dot(h, whh_t, preferred_element_type=jnp.float32))
    i = jax.nn.sigmoid(g[:, 0:D])
    f = jax.nn.sigmoid(g[:, D:2 * D])
    gg = jnp.tanh(g[:, 2 * D:3 * D])
    o = jax.nn.sigmoid(g[:, 3 * D:4 * D])
    cn = f * c + i * gg
    hn = o * jnp.tanh(cn)
    return hn, cn, hn + RESIDUAL_WEIGHT * inp


def _layer_body(wa, wb, gb, h, c, inp, wih_t, whh_t, wnt, e1, an,
                h_o, c_o, x_o, xt_o, aux_o):
    hn, cn, xn = _lstm(wa[...], wb[...], gb[...], h[...], c[...], inp[...],
                       wih_t[...], whh_t[...])
    h_o[...] = hn
    c_o[...] = cn
    x_o[...] = xn
    xt = jnp.dot(xn, wnt[...], preferred_element_type=jnp.float32) + e1[...]
    xt_o[...] = xt
    aux_o[...] = jnp.dot(xt, an[...], preferred_element_type=jnp.float32)


_tc_layer = pl.pallas_call(
    _layer_body,
    grid=(GRID,),
    in_specs=[_rows(WIDE), _rows(WIDE), _full((1, D)), _rows(D), _rows(D),
              _rows(D), _full((D, 4 * D)), _full((D, 4 * D)),
              _full((D, WIDE)), _full((1, WIDE)), _full((WIDE, D))],
    out_specs=[_rows(D), _rows(D), _rows(D), _rows(WIDE), _rows(D)],
    out_shape=[jax.ShapeDtypeStruct((N, D), jnp.float32),
               jax.ShapeDtypeStruct((N, D), jnp.float32),
               jax.ShapeDtypeStruct((N, D), jnp.float32),
               jax.ShapeDtypeStruct((N, WIDE), jnp.float32),
               jax.ShapeDtypeStruct((N, D), jnp.float32)],
)


def _final_body(wa, wb, gb, h, c, inp, wih_t, whh_t, w2t, b2, out_o):
    _, _, xn = _lstm(wa[...], wb[...], gb[...], h[...], c[...], inp[...],
                     wih_t[...], whh_t[...])
    out_o[...] = (jnp.dot(xn, w2t[...], preferred_element_type=jnp.float32)
                  + b2[...])


_tc_final = pl.pallas_call(
    _final_body,
    grid=(GRID,),
    in_specs=[_rows(WIDE), _rows(WIDE), _full((1, D)), _rows(D), _rows(D),
              _rows(D), _full((D, 4 * D)), _full((D, 4 * D)),
              _full((D, D)), _full((1, D))],
    out_specs=_rows(D),
    out_shape=jax.ShapeDtypeStruct((N, D), jnp.float32),
)


def kernel(x, edge_index, lin1_W, lin1_b, gat_W, att_src, att_dst, gat_b,
           Wih, Whh, lin2_W, lin2_b):
    loops = jnp.arange(N, dtype=edge_index.dtype)
    pad = jnp.zeros((EP - NE,), edge_index.dtype)
    src = jnp.concatenate([edge_index[0], loops, pad])
    dst = jnp.concatenate([edge_index[1], loops, pad])
    pk = jnp.stack([src.reshape(EP // C, NSUB, S),
                    dst.reshape(EP // C, NSUB, S)], axis=1)

    W144 = jnp.zeros((L, D, WIDE), jnp.float32)
    W144 = W144.at[:, :, :D].set(jnp.transpose(gat_W, (0, 2, 1)))
    e1 = jnp.zeros((1, WIDE), jnp.float32).at[0, D].set(1.0)
    A = jnp.zeros((L, WIDE, D), jnp.float32)
    A = A.at[:, :D, 0].set(att_src).at[:, :D, 1].set(att_dst)
    Wih_t = jnp.transpose(Wih, (0, 2, 1))
    Whh_t = jnp.transpose(Whh, (0, 2, 1))

    inp, xt, aux = _tc_prelude(x, lin1_W.T, lin1_b.reshape(1, D),
                               W144[0], e1, A[0])
    h = jnp.zeros((N, D), jnp.float32)
    c = jnp.zeros((N, D), jnp.float32)
    for l in range(L):
        wides = _sc_edge()(xt, aux[:, 0], aux[:, 1], pk)[:, :N]
        gb = gat_b[l].reshape(1, D)
        if l < L - 1:
            h, c, _x, xt, aux = _tc_layer(
                wides[0], wides[1], gb, h, c, inp, Wih_t[l], Whh_t[l],
                W144[l + 1], e1, A[l + 1])
        else:
            return _tc_final(wides[0], wides[1], gb, h, c, inp,
                             Wih_t[l], Whh_t[l], lin2_W.T,
                             lin2_b.reshape(1, D))


# speedup vs baseline: 29.6386x; 1.0927x over previous
import functools

import jax
import jax.numpy as jnp
from jax import lax
from jax.experimental import pallas as pl
from jax.experimental.pallas import tpu as pltpu
from jax.experimental.pallas import tpu_sc as plsc

N, E, D, L = 10000, 320000, 128, 3
RESIDUAL_WEIGHT = 0.1
NE = E + N
S = 32
NSUB = 3
C = S * NSUB
NTILES = 32
EP = ((NE + NTILES * C - 1) // (NTILES * C)) * (NTILES * C)
EPT = EP // NTILES
NCH = EPT // C
WIDE = 144
NP = 10240
RPT = NP // 16
BN = 1000
GRID = N // BN
_RCHUNKS = [(t * S, S) for t in range(RPT // S)]
if RPT % S:
    _RCHUNKS.append((RPT // S * S, RPT % S))


def _sc_edge_body(xt_hbm, as_hbm, ad_hbm, idx_hbm, out_hbm,
                  as_t, ad_t, idx, rows, wbuf, table,
                  gsem0, gsem1, gsem2, ssem0, ssem1, ssem2, isem):
    cid = lax.axis_index("c")
    sid = lax.axis_index("s")
    wid = cid * 16 + sid
    gsem = (gsem0, gsem1, gsem2)
    ssem = (ssem0, ssem1, ssem2)

    pltpu.sync_copy(as_hbm, as_t)
    pltpu.sync_copy(ad_hbm, ad_t)

    z16 = jnp.zeros((16,), jnp.float32)

    def _zrow(i, _):
        for k in range(WIDE // 16):
            rows[0, i, pl.ds(k * 16, 16)] = z16
        return 0

    lax.fori_loop(0, S, _zrow, 0)
    r0 = sid * RPT
    for off, size in _RCHUNKS:
        pltpu.sync_copy(rows.at[0, pl.ds(0, size)],
                        table.at[pl.ds(r0 + off, size)])
    plsc.subcore_barrier()

    kk0 = wid * NCH
    base = wid * EPT
    lanes = lax.iota(jnp.int32, 16)

    def _gissue(p, s):
        pltpu.async_copy(xt_hbm.at[idx.at[p, 0, s]], rows.at[s], gsem[s])

    def _gwait(p, s):
        pltpu.make_async_copy(xt_hbm.at[idx.at[p, 0, s]], rows.at[s],
                              gsem[s]).wait()

    def _sissue(p, s):
        pltpu.async_copy(rows.at[s], table.at[idx.at[p, 1, s]], ssem[s],
                         add=True)

    def _swait(p, s):
        pltpu.make_async_copy(rows.at[s], table.at[idx.at[p, 1, s]],
                              ssem[s]).wait()

    def _proc(ci, p, s):
        ebase = base + ci * C + s * S
        for j in range(S // 16):
            s_ids = idx[p, 0, s, pl.ds(j * 16, 16)]
            d_ids = idx[p, 1, s, pl.ds(j * 16, 16)]
            e = (plsc.load_gather(as_t, [s_ids])
                 + plsc.load_gather(ad_t, [d_ids]))
            e = jnp.where(e >= 0.0, e, 0.2 * e)
            w = jnp.exp(e)
            w = jnp.where(ebase + j * 16 + lanes < NE, w, 0.0)
            wbuf[pl.ds(j * 16, 16)] = w

        def _srow(rr, _):
            r0 = rr * 2
            r1 = r0 + 1
            wv0 = plsc.load_gather(wbuf, [jnp.full((16,), r0, jnp.int32)])
            wv1 = plsc.load_gather(wbuf, [jnp.full((16,), r1, jnp.int32)])
            for k in range(D // 16):
                rows[s, r0, pl.ds(k * 16, 16)] = (
                    rows[s, r0, pl.ds(k * 16, 16)] * wv0)
                rows[s, r1, pl.ds(k * 16, 16)] = (
                    rows[s, r1, pl.ds(k * 16, 16)] * wv1)
            rows[s, r0, pl.ds(D, 16)] = wv0
            rows[s, r1, pl.ds(D, 16)] = wv1
            return 0

        lax.fori_loop(0, S // 2, _srow, 0)

    def _chunk(ci, p, first=False, last=False):
        _gwait(p, 0)
        _proc(ci, p, 0)
        _sissue(p, 0)
        if not first:
            _swait(1 - p, 2)
        if not last:
            pltpu.async_copy(idx_hbm.at[kk0 + ci + 1], idx.at[1 - p], isem)
        _gissue(p, 2)
        _gwait(p, 1)
        _proc(ci, p, 1)
        _sissue(p, 1)
        _swait(p, 0)
        if not last:
            pltpu.make_async_copy(idx_hbm.at[kk0 + ci + 1], idx.at[1 - p],
                                  isem).wait()
            _gissue(1 - p, 0)
        _gwait(p, 2)
        _proc(ci, p, 2)
        _sissue(p, 2)
        _swait(p, 1)
        if not last:
            _gissue(1 - p, 1)

    pltpu.sync_copy(idx_hbm.at[kk0], idx.at[0])
    _gissue(0, 0)
    _gissue(0, 1)
    _chunk(0, 0, first=True)

    def _outer(cc, _):
        ci = 1 + 2 * cc
        _chunk(ci, 1)
        _chunk(ci + 1, 0)
        return 0

    lax.fori_loop(0, (NCH - 2) // 2, _outer, 0)
    _chunk(NCH - 1, 1, last=True)
    _swait(1, 2)
    plsc.subcore_barrier()

    for off, size in _RCHUNKS:
        pltpu.sync_copy(table.at[pl.ds(r0 + off, size)],
                        out_hbm.at[cid, pl.ds(r0 + off, size)])


@functools.cache
def _sc_edge():
    return pl.kernel(
        _sc_edge_body,
        out_type=jax.ShapeDtypeStruct((2, NP, WIDE), jnp.float32),
        mesh=plsc.VectorSubcoreMesh(core_axis_name="c", subcore_axis_name="s"),
        compiler_params=pltpu.CompilerParams(needs_layout_passes=False,
                                             use_tc_tiling_on_sc=False),
        scratch_types=[
            pltpu.VMEM((N,), jnp.float32),
            pltpu.VMEM((N,), jnp.float32),
            pltpu.VMEM((2, 2, NSUB, S), jnp.int32),
            pltpu.VMEM((NSUB, S, WIDE), jnp.float32),
            pltpu.VMEM((S,), jnp.float32),
            pltpu.VMEM_SHARED((NP, WIDE), jnp.float32),
            pltpu.SemaphoreType.DMA,
            pltpu.SemaphoreType.DMA,
            pltpu.SemaphoreType.DMA,
            pltpu.SemaphoreType.DMA,
            pltpu.SemaphoreType.DMA,
            pltpu.SemaphoreType.DMA,
            pltpu.SemaphoreType.DMA,
        ],
    )


def _full(shape):
    return pl.BlockSpec(shape, lambda i: (0,) * len(shape))


def _rows(width):
    return pl.BlockSpec((BN, width), lambda i: (i, 0))


def _prelude_body(x_ref, w1t, b1, w0t, e1, a0, x1_o, xt_o, aux_o):
    xr = jnp.dot(x_ref[...], w1t[...], preferred_element_type=jnp.float32) + b1[...]
    x1_o[...] = xr
    xt = jnp.dot(xr, w0t[...], preferred_element_type=jnp.float32) + e1[...]
    xt_o[...] = xt
    aux_o[...] = jnp.dot(xt, a0[...], preferred_element_type=jnp.float32)


_tc_prelude = pl.pallas_call(
    _prelude_body,
    grid=(GRID,),
    in_specs=[_rows(D), _full((D, D)), _full((1, D)), _full((D, WIDE)),
              _full((1, WIDE)), _full((WIDE, D))],
    out_specs=[_rows(D), _rows(WIDE), _rows(D)],
    out_shape=[jax.ShapeDtypeStruct((N, D), jnp.float32),
               jax.ShapeDtypeStruct((N, WIDE), jnp.float32),
               jax.ShapeDtypeStruct((N, D), jnp.float32)],
)


def _lstm(wa, wb, gb, h, c, inp, wih_t, whh_t):
    den = wa[:, D:D + 1] + wb[:, D:D + 1]
    num = wa[:, :D] + wb[:, :D]
    xb = jnp.tanh(num / den + gb)
    g = (jnp.dot(xb, wih_t, preferred_element_type=jnp.float32)
         + jnp.dot(h, whh_t, preferred_element_type=jnp.float32))
    i = jax.nn.sigmoid(g[:, 0:D])
    f = jax.nn.sigmoid(g[:, D:2 * D])
    gg = jnp.tanh(g[:, 2 * D:3 * D])
    o = jax.nn.sigmoid(g[:, 3 * D:4 * D])
    cn = f * c + i * gg
    hn = o * jnp.tanh(cn)
    return hn, cn, hn + RESIDUAL_WEIGHT * inp


def _layer_body(wa, wb, gb, h, c, inp, wih_t, whh_t, wnt, e1, an,
                h_o, c_o, x_o, xt_o, aux_o):
    hn, cn, xn = _lstm(wa[...], wb[...], gb[...], h[...], c[...], inp[...],
                       wih_t[...], whh_t[...])
    h_o[...] = hn
    c_o[...] = cn
    x_o[...] = xn
    xt = jnp.dot(xn, wnt[...], preferred_element_type=jnp.float32) + e1[...]
    xt_o[...] = xt
    aux_o[...] = jnp.dot(xt, an[...], preferred_element_type=jnp.float32)


_tc_layer = pl.pallas_call(
    _layer_body,
    grid=(GRID,),
    in_specs=[_rows(WIDE), _rows(WIDE), _full((1, D)), _rows(D), _rows(D),
              _rows(D), _full((D, 4 * D)), _full((D, 4 * D)),
              _full((D, WIDE)), _full((1, WIDE)), _full((WIDE, D))],
    out_specs=[_rows(D), _rows(D), _rows(D), _rows(WIDE), _rows(D)],
    out_shape=[jax.ShapeDtypeStruct((N, D), jnp.float32),
               jax.ShapeDtypeStruct((N, D), jnp.float32),
               jax.ShapeDtypeStruct((N, D), jnp.float32),
               jax.ShapeDtypeStruct((N, WIDE), jnp.float32),
               jax.ShapeDtypeStruct((N, D), jnp.float32)],
)


def _final_body(wa, wb, gb, h, c, inp, wih_t, whh_t, w2t, b2, out_o):
    _, _, xn = _lstm(wa[...], wb[...], gb[...], h[...], c[...], inp[...],
                     wih_t[...], whh_t[...])
    out_o[...] = (jnp.dot(xn, w2t[...], preferred_element_type=jnp.float32)
                  + b2[...])


_tc_final = pl.pallas_call(
    _final_body,
    grid=(GRID,),
    in_specs=[_rows(WIDE), _rows(WIDE), _full((1, D)), _rows(D), _rows(D),
              _rows(D), _full((D, 4 * D)), _full((D, 4 * D)),
              _full((D, D)), _full((1, D))],
    out_specs=_rows(D),
    out_shape=jax.ShapeDtypeStruct((N, D), jnp.float32),
)


def kernel(x, edge_index, lin1_W, lin1_b, gat_W, att_src, att_dst, gat_b,
           Wih, Whh, lin2_W, lin2_b):
    loops = jnp.arange(N, dtype=edge_index.dtype)
    pad = jnp.zeros((EP - NE,), edge_index.dtype)
    src = jnp.concatenate([edge_index[0], loops, pad])
    dst = jnp.concatenate([edge_index[1], loops, pad])
    pk = jnp.stack([src.reshape(EP // C, NSUB, S),
                    dst.reshape(EP // C, NSUB, S)], axis=1)

    W144 = jnp.zeros((L, D, WIDE), jnp.float32)
    W144 = W144.at[:, :, :D].set(jnp.transpose(gat_W, (0, 2, 1)))
    e1 = jnp.zeros((1, WIDE), jnp.float32).at[0, D].set(1.0)
    A = jnp.zeros((L, WIDE, D), jnp.float32)
    A = A.at[:, :D, 0].set(att_src).at[:, :D, 1].set(att_dst)
    Wih_t = jnp.transpose(Wih, (0, 2, 1))
    Whh_t = jnp.transpose(Whh, (0, 2, 1))

    inp, xt, aux = _tc_prelude(x, lin1_W.T, lin1_b.reshape(1, D),
                               W144[0], e1, A[0])
    h = jnp.zeros((N, D), jnp.float32)
    c = jnp.zeros((N, D), jnp.float32)
    for l in range(L):
        wides = _sc_edge()(xt, aux[:, 0], aux[:, 1], pk)[:, :N]
        gb = gat_b[l].reshape(1, D)
        if l < L - 1:
            h, c, _x, xt, aux = _tc_layer(
                wides[0], wides[1], gb, h, c, inp, Wih_t[l], Whh_t[l],
                W144[l + 1], e1, A[l + 1])
        else:
            return _tc_final(wides[0], wides[1], gb, h, c, inp,
                             Wih_t[l], Whh_t[l], lin2_W.T,
                             lin2_b.reshape(1, D))


# speedup vs baseline: 30.4381x; 1.0270x over previous
import functools

import jax
import jax.numpy as jnp
from jax import lax
from jax.experimental import pallas as pl
from jax.experimental.pallas import tpu as pltpu
from jax.experimental.pallas import tpu_sc as plsc

N, E, D, L = 10000, 320000, 128, 3
RESIDUAL_WEIGHT = 0.1
NE = E + N
S = 32
NSUB = 3
C = S * NSUB
NTILES = 32
EP = ((NE + NTILES * C - 1) // (NTILES * C)) * (NTILES * C)
EPT = EP // NTILES
NCH = EPT // C
WIDE = 144
NP = 10240
RPT = NP // 16
BN = 1000
GRID = N // BN
_RCHUNKS = [(t * S, S) for t in range(RPT // S)]
if RPT % S:
    _RCHUNKS.append((RPT // S * S, RPT % S))


def _sc_edge_body(xt_hbm, as_hbm, ad_hbm, idx_hbm, out_hbm,
                  as_t, ad_t, idx, rows, wbuf, table,
                  gsem0, gsem1, gsem2, ssem0, ssem1, ssem2, isem):
    cid = lax.axis_index("c")
    sid = lax.axis_index("s")
    wid = cid * 16 + sid
    gsem = (gsem0, gsem1, gsem2)
    ssem = (ssem0, ssem1, ssem2)

    pltpu.sync_copy(as_hbm, as_t)
    pltpu.sync_copy(ad_hbm, ad_t)

    z16 = jnp.zeros((16,), jnp.float32)

    def _zrow(i, _):
        for k in range(WIDE // 16):
            rows[0, i, pl.ds(k * 16, 16)] = z16
        return 0

    lax.fori_loop(0, S, _zrow, 0)
    r0 = sid * RPT
    for off, size in _RCHUNKS:
        pltpu.sync_copy(rows.at[0, pl.ds(0, size)],
                        table.at[pl.ds(r0 + off, size)])
    plsc.subcore_barrier()

    kk0 = wid * NCH
    base = wid * EPT
    lanes = lax.iota(jnp.int32, 16)

    def _gissue(p, s):
        pltpu.async_copy(xt_hbm.at[idx.at[p, 0, s]], rows.at[s], gsem[s])

    def _gwait(p, s):
        pltpu.make_async_copy(xt_hbm.at[idx.at[p, 0, s]], rows.at[s],
                              gsem[s]).wait()

    def _sissue(p, s):
        pltpu.async_copy(rows.at[s], table.at[idx.at[p, 1, s]], ssem[s],
                         add=True)

    def _swait(p, s):
        pltpu.make_async_copy(rows.at[s], table.at[idx.at[p, 1, s]],
                              ssem[s]).wait()

    def _proc(ci, p, s):
        ebase = base + ci * C + s * S
        for j in range(S // 16):
            s_ids = idx[p, 0, s, pl.ds(j * 16, 16)]
            d_ids = idx[p, 1, s, pl.ds(j * 16, 16)]
            e = (plsc.load_gather(as_t, [s_ids])
                 + plsc.load_gather(ad_t, [d_ids]))
            e = jnp.where(e >= 0.0, e, 0.2 * e)
            w = jnp.exp(e)
            w = jnp.where(ebase + j * 16 + lanes < NE, w, 0.0)
            wbuf[pl.ds(j * 16, 16)] = w

        def _srow(rr, _):
            r0 = rr * 2
            r1 = r0 + 1
            wv0 = plsc.load_gather(wbuf, [jnp.full((16,), r0, jnp.int32)])
            wv1 = plsc.load_gather(wbuf, [jnp.full((16,), r1, jnp.int32)])
            for k in range(D // 16):
                rows[s, r0, pl.ds(k * 16, 16)] = (
                    rows[s, r0, pl.ds(k * 16, 16)] * wv0)
                rows[s, r1, pl.ds(k * 16, 16)] = (
                    rows[s, r1, pl.ds(k * 16, 16)] * wv1)
            rows[s, r0, pl.ds(D, 16)] = wv0
            rows[s, r1, pl.ds(D, 16)] = wv1
            return 0

        lax.fori_loop(0, S // 2, _srow, 0)

    def _chunk(ci, p, first=False, last=False):
        _gwait(p, 0)
        _proc(ci, p, 0)
        _sissue(p, 0)
        if not first:
            _swait(1 - p, 2)
        if not last:
            pltpu.async_copy(idx_hbm.at[kk0 + ci + 1], idx.at[1 - p], isem)
        _gissue(p, 2)
        _gwait(p, 1)
        _proc(ci, p, 1)
        _sissue(p, 1)
        _swait(p, 0)
        if not last:
            pltpu.make_async_copy(idx_hbm.at[kk0 + ci + 1], idx.at[1 - p],
                                  isem).wait()
            _gissue(1 - p, 0)
        _gwait(p, 2)
        _proc(ci, p, 2)
        _sissue(p, 2)
        _swait(p, 1)
        if not last:
            _gissue(1 - p, 1)

    pltpu.sync_copy(idx_hbm.at[kk0], idx.at[0])
    _gissue(0, 0)
    _gissue(0, 1)
    _chunk(0, 0, first=True)

    def _outer(cc, _):
        ci = 1 + 2 * cc
        _chunk(ci, 1)
        _chunk(ci + 1, 0)
        return 0

    lax.fori_loop(0, (NCH - 2) // 2, _outer, 0)
    _chunk(NCH - 1, 1, last=True)
    _swait(1, 2)
    plsc.subcore_barrier()

    for off, size in _RCHUNKS:
        pltpu.sync_copy(table.at[pl.ds(r0 + off, size)],
                        out_hbm.at[cid, pl.ds(r0 + off, size)])


@functools.cache
def _sc_edge():
    return pl.kernel(
        _sc_edge_body,
        out_type=jax.ShapeDtypeStruct((2, NP, WIDE), jnp.float32),
        mesh=plsc.VectorSubcoreMesh(core_axis_name="c", subcore_axis_name="s"),
        compiler_params=pltpu.CompilerParams(needs_layout_passes=False,
                                             use_tc_tiling_on_sc=False),
        scratch_types=[
            pltpu.VMEM((N,), jnp.float32),
            pltpu.VMEM((N,), jnp.float32),
            pltpu.VMEM((2, 2, NSUB, S), jnp.int32),
            pltpu.VMEM((NSUB, S, WIDE), jnp.float32),
            pltpu.VMEM((S,), jnp.float32),
            pltpu.VMEM_SHARED((NP, WIDE), jnp.float32),
            pltpu.SemaphoreType.DMA,
            pltpu.SemaphoreType.DMA,
            pltpu.SemaphoreType.DMA,
            pltpu.SemaphoreType.DMA,
            pltpu.SemaphoreType.DMA,
            pltpu.SemaphoreType.DMA,
            pltpu.SemaphoreType.DMA,
        ],
    )


def _full(shape):
    return pl.BlockSpec(shape, lambda i: (0,) * len(shape))


def _rows(width):
    return pl.BlockSpec((BN, width), lambda i: (i, 0))


def _col():
    return pl.BlockSpec((BN, 1), lambda i: (i, 0))


_CD1 = (((1,), (1,)), ((), ()))


def _dott(a, b):
    return lax.dot_general(a, b, _CD1, preferred_element_type=jnp.float32)


def _emit_gat_inputs(xn, wn, asr, adr, xt_o, as_o, ad_o):
    xt = _dott(xn, wn)
    xt_o[:, :D] = xt
    xt_o[:, D:] = (lax.broadcasted_iota(jnp.int32, (BN, WIDE - D), 1) == 0
                   ).astype(jnp.float32)
    as_o[...] = jnp.dot(xt, asr, preferred_element_type=jnp.float32)[:, None]
    ad_o[...] = jnp.dot(xt, adr, preferred_element_type=jnp.float32)[:, None]


def _prelude_body(x_ref, w1, b1, w0, asr, adr, x1_o, xt_o, as_o, ad_o):
    xr = _dott(x_ref[...], w1[...]) + b1[...]
    x1_o[...] = xr
    _emit_gat_inputs(xr, w0[0], asr[0, 0], adr[0, 0], xt_o, as_o, ad_o)


_tc_prelude = pl.pallas_call(
    _prelude_body,
    grid=(GRID,),
    in_specs=[_rows(D), _full((D, D)), _full((D,)),
              pl.BlockSpec((1, D, D), lambda i: (0, 0, 0)),
              pl.BlockSpec((1, 1, D), lambda i: (0, 0, 0)),
              pl.BlockSpec((1, 1, D), lambda i: (0, 0, 0))],
    out_specs=[_rows(D), _rows(WIDE), _col(), _col()],
    out_shape=[jax.ShapeDtypeStruct((N, D), jnp.float32),
               jax.ShapeDtypeStruct((N, WIDE), jnp.float32),
               jax.ShapeDtypeStruct((N, 1), jnp.float32),
               jax.ShapeDtypeStruct((N, 1), jnp.float32)],
)


def _lstm(wides, gb, h, c, inp, wih, whh):
    wa = wides[0]
    wb = wides[1]
    den = wa[:, D:D + 1] + wb[:, D:D + 1]
    num = wa[:, :D] + wb[:, :D]
    xb = jnp.tanh(num / den + gb)
    g = _dott(xb, wih) + _dott(h, whh)
    i = jax.nn.sigmoid(g[:, 0:D])
    f = jax.nn.sigmoid(g[:, D:2 * D])
    gg = jnp.tanh(g[:, 2 * D:3 * D])
    o = jax.nn.sigmoid(g[:, 3 * D:4 * D])
    cn = f * c + i * gg
    hn = o * jnp.tanh(cn)
    return hn, cn, hn + RESIDUAL_WEIGHT * inp


def _layer_body(wides, gbr, h, c, inp, wih, whh, wn, asr, adr,
                h_o, c_o, x_o, xt_o, as_o, ad_o):
    hn, cn, xn = _lstm(wides[...], gbr[0], h[...], c[...], inp[...],
                       wih[0], whh[0])
    h_o[...] = hn
    c_o[...] = cn
    x_o[...] = xn
    _emit_gat_inputs(xn, wn[0], asr[0, 0], adr[0, 0], xt_o, as_o, ad_o)


@functools.cache
def _tc_layer(l):
    return pl.pallas_call(
        _layer_body,
        grid=(GRID,),
        in_specs=[pl.BlockSpec((2, BN, WIDE), lambda i: (0, i, 0)),
                  pl.BlockSpec((1, 1, D), lambda i: (l, 0, 0)),
                  _rows(D), _rows(D), _rows(D),
                  pl.BlockSpec((1, 4 * D, D), lambda i: (l, 0, 0)),
                  pl.BlockSpec((1, 4 * D, D), lambda i: (l, 0, 0)),
                  pl.BlockSpec((1, D, D), lambda i: (l + 1, 0, 0)),
                  pl.BlockSpec((1, 1, D), lambda i: (l + 1, 0, 0)),
                  pl.BlockSpec((1, 1, D), lambda i: (l + 1, 0, 0))],
        out_specs=[_rows(D), _rows(D), _rows(D), _rows(WIDE), _col(), _col()],
        out_shape=[jax.ShapeDtypeStruct((N, D), jnp.float32),
                   jax.ShapeDtypeStruct((N, D), jnp.float32),
                   jax.ShapeDtypeStruct((N, D), jnp.float32),
                   jax.ShapeDtypeStruct((N, WIDE), jnp.float32),
                   jax.ShapeDtypeStruct((N, 1), jnp.float32),
                   jax.ShapeDtypeStruct((N, 1), jnp.float32)],
    )


def _final_body(wides, gbr, h, c, inp, wih, whh, w2, b2, out_o):
    _, _, xn = _lstm(wides[...], gbr[0], h[...], c[...], inp[...],
                     wih[0], whh[0])
    out_o[...] = _dott(xn, w2[...]) + b2[...]


_tc_final = pl.pallas_call(
    _final_body,
    grid=(GRID,),
    in_specs=[pl.BlockSpec((2, BN, WIDE), lambda i: (0, i, 0)),
              pl.BlockSpec((1, 1, D), lambda i: (L - 1, 0, 0)),
              _rows(D), _rows(D), _rows(D),
              pl.BlockSpec((1, 4 * D, D), lambda i: (L - 1, 0, 0)),
              pl.BlockSpec((1, 4 * D, D), lambda i: (L - 1, 0, 0)),
              _full((D, D)), _full((D,))],
    out_specs=_rows(D),
    out_shape=jax.ShapeDtypeStruct((N, D), jnp.float32),
)


def kernel(x, edge_index, lin1_W, lin1_b, gat_W, att_src, att_dst, gat_b,
           Wih, Whh, lin2_W, lin2_b):
    loops = jnp.arange(N, dtype=edge_index.dtype)
    pad = jnp.zeros((EP - NE,), edge_index.dtype)
    src = jnp.concatenate([edge_index[0], loops, pad])
    dst = jnp.concatenate([edge_index[1], loops, pad])
    pk = jnp.stack([src.reshape(EP // C, NSUB, S),
                    dst.reshape(EP // C, NSUB, S)], axis=1)

    asr = att_src.reshape(L, 1, D)
    adr = att_dst.reshape(L, 1, D)
    gbr = gat_b.reshape(L, 1, D)

    inp, xt, a_s, a_d = _tc_prelude(x, lin1_W, lin1_b, gat_W, asr, adr)
    h = jnp.zeros((N, D), jnp.float32)
    c = jnp.zeros((N, D), jnp.float32)
    for l in range(L):
        wides = _sc_edge()(xt, a_s.reshape(N), a_d.reshape(N), pk)
        if l < L - 1:
            h, c, _x, xt, a_s, a_d = _tc_layer(l)(
                wides, gbr, h, c, inp, Wih, Whh, gat_W, asr, adr)
        else:
            return _tc_final(wides, gbr, h, c, inp, Wih, Whh,
                             lin2_W, lin2_b)
